# Initial kernel scaffold; baseline (speedup 1.0000x reference)
import jax
import jax.numpy as jnp
from jax.experimental import pallas as pl


def kernel(x, edges, local_graph_weight, global_mask, params):
    raise NotImplementedError("write your pallas kernel here")


# speedup vs baseline: 153.0155x; 153.0155x over previous
import functools
import numpy as np
import jax
import jax.numpy as jnp
from jax import lax
from jax.experimental import pallas as pl
from jax.experimental.pallas import tpu as pltpu
from jax.experimental.pallas import tpu_sc as plsc

N_NODES = 64
N_EDGES = 4096
N_FULL = N_EDGES + N_NODES
B = 256
W_IN = 657
W_OUT = 41

_DB4_LO = np.array([
    -0.010597401784997278, 0.032883011666982945, 0.030841381835986965,
    -0.18703481171888114, -0.02798376941698385, 0.6308807679295904,
    0.7148465705525415, 0.23037781330885523], dtype=np.float64)


def _lo_mat(L):
    Lp = (L - 2) // 2 + 1
    M = np.zeros((L, Lp), np.float64)
    for i in range(Lp):
        for k in range(8):
            M[(2 * i + k - 3) % L, i] += _DB4_LO[7 - k]
    return M


_M_TOTAL = np.asarray(
    _lo_mat(657) @ _lo_mat(328) @ _lo_mat(164) @ _lo_mat(82),
    dtype=np.float32)

_r = np.arange(N_EDGES) // 64
_c = np.arange(N_EDGES) % 64
_mx = np.maximum(_r, _c)
_mn = np.minimum(_r, _c)
_GIDX = np.asarray(_mx * (_mx + 1) // 2 + _mn, dtype=np.int32)

_EYES = np.stack([np.eye(64, k=o, dtype=np.float32) for o in range(-3, 4)])


def _rsqrt16(x):
    i = plsc.bitcast(x, jnp.int32)
    i = jnp.int32(0x5F3759DF) - (i >> 1)
    y = plsc.bitcast(i, jnp.float32)
    xh = x * jnp.float32(0.5)
    for _ in range(4):
        y = y * (jnp.float32(1.5) - xh * y * y)
    return y


def _adj_body(src_h, dst_h, wl_h, aw_h, gi_h, mv_h, al_h, ag_h,
              src_v, dst_v, wl_v, wg_v, aw_v, gi_v, mv_v,
              deg_v, dinv_v, acc_v, sem):
    first = jnp.logical_and(lax.axis_index("c") == 0, lax.axis_index("s") == 0)

    @pl.when(first)
    def _():
        pltpu.sync_copy(src_h, src_v)
        pltpu.sync_copy(dst_h, dst_v)
        pltpu.sync_copy(wl_h, wl_v)
        pltpu.sync_copy(aw_h, aw_v)
        pltpu.sync_copy(gi_h, gi_v)
        pltpu.sync_copy(mv_h, mv_v)

        def gw_body(i, _):
            s = pl.ds(i * 16, 16)
            g = plsc.load_gather(aw_v, [gi_v[s]])
            wg_v[s] = g * mv_v[s]
            return _
        lax.fori_loop(0, N_FULL // 16, gw_body, None)

        def one_chain(w_ref, out_h):
            def z_deg(i, _):
                deg_v[pl.ds(i * 16, 16)] = jnp.zeros((16,), jnp.float32)
                return _
            lax.fori_loop(0, N_NODES // 16, z_deg, None)

            def z_acc(i, _):
                acc_v[pl.ds(i * 16, 16)] = jnp.zeros((16,), jnp.float32)
                return _
            lax.fori_loop(0, (N_NODES * N_NODES) // 16, z_acc, None)

            def deg_body(i, _):
                s = pl.ds(i * 16, 16)
                plsc.addupdate_scatter(deg_v, [dst_v[s]], jnp.abs(w_ref[s]))
                return _
            lax.fori_loop(0, N_FULL // 16, deg_body, None)

            def dinv_body(i, _):
                s = pl.ds(i * 16, 16)
                d = deg_v[s]
                r = _rsqrt16(jnp.maximum(d, jnp.float32(1e-12)))
                dinv_v[s] = jnp.where(d > 0, r, jnp.float32(0.0))
                return _
            lax.fori_loop(0, N_NODES // 16, dinv_body, None)

            def edge_body(i, _):
                s = pl.ds(i * 16, 16)
                sv = src_v[s]
                dv = dst_v[s]
                ds_ = plsc.load_gather(dinv_v, [sv])
                dd = plsc.load_gather(dinv_v, [dv])
                norm = ds_ * w_ref[s] * dd
                flat = dv * jnp.int32(64) + sv
                plsc.addupdate_scatter(acc_v, [flat], norm)
                return _
            lax.fori_loop(0, N_FULL // 16, edge_body, None)

            pltpu.sync_copy(acc_v, out_h)

        one_chain(wl_v, al_h)
        one_chain(wg_v, ag_h)


def _build_adj(src_f, dst_f, wloc, awx, gidx, mvec):
    mesh = plsc.VectorSubcoreMesh(core_axis_name="c", subcore_axis_name="s")
    f = pl.kernel(
        _adj_body,
        mesh=mesh,
        compiler_params=pltpu.CompilerParams(needs_layout_passes=False),
        out_type=[
            jax.ShapeDtypeStruct((N_NODES * N_NODES,), jnp.float32),
            jax.ShapeDtypeStruct((N_NODES * N_NODES,), jnp.float32),
        ],
        scratch_types=[
            pltpu.VMEM((N_FULL,), jnp.int32),
            pltpu.VMEM((N_FULL,), jnp.int32),
            pltpu.VMEM((N_FULL,), jnp.float32),
            pltpu.VMEM((N_FULL,), jnp.float32),
            pltpu.VMEM((2088,), jnp.float32),
            pltpu.VMEM((N_FULL,), jnp.int32),
            pltpu.VMEM((N_FULL,), jnp.float32),
            pltpu.VMEM((N_NODES,), jnp.float32),
            pltpu.VMEM((N_NODES,), jnp.float32),
            pltpu.VMEM((N_NODES * N_NODES,), jnp.float32),
            pltpu.SemaphoreType.DMA,
        ],
    )
    return f(src_f, dst_f, wloc, awx, gidx, mvec)


def _s1_body(x_ref, w1t_ref, w2t_ref, sa_ref, sm_ref, m_ref, out_ref):
    x = x_ref[...]
    avg = jnp.mean(x, axis=1)
    mx = jnp.max(x, axis=1)
    w1t = w1t_ref[...]
    w2t = w2t_ref[...]
    h = jnp.maximum(jnp.dot(avg, w1t, preferred_element_type=jnp.float32), 0.0)
    att = jnp.dot(h, w2t, preferred_element_type=jnp.float32)
    h = jnp.maximum(jnp.dot(mx, w1t, preferred_element_type=jnp.float32), 0.0)
    att = att + jnp.dot(h, w2t, preferred_element_type=jnp.float32)
    att = jax.nn.sigmoid(att)
    x1 = x * att[:, None, :]
    a = jnp.mean(x1, axis=2)
    m = jnp.max(x1, axis=2)
    sa = (jnp.dot(a, sa_ref[...], preferred_element_type=jnp.float32)
          + jnp.dot(m, sm_ref[...], preferred_element_type=jnp.float32))
    x2 = x1 * jax.nn.sigmoid(sa)[:, :, None]
    bB = x2.shape[0]
    d = jnp.dot(x2.reshape(bB * N_NODES, W_IN), m_ref[...],
                preferred_element_type=jnp.float32)
    out_ref[...] = d.reshape(bB, N_NODES, W_OUT)


def _stage1(xs, w1t, w2t, s_a, s_m, mmat, bB=32):
    grid = (B // bB,)
    return pl.pallas_call(
        _s1_body,
        grid=grid,
        in_specs=[
            pl.BlockSpec((bB, N_NODES, W_IN), lambda i: (i, 0, 0)),
            pl.BlockSpec((W_IN, 82), lambda i: (0, 0)),
            pl.BlockSpec((82, W_IN), lambda i: (0, 0)),
            pl.BlockSpec((N_NODES, N_NODES), lambda i: (0, 0)),
            pl.BlockSpec((N_NODES, N_NODES), lambda i: (0, 0)),
            pl.BlockSpec((W_IN, W_OUT), lambda i: (0, 0)),
        ],
        out_specs=pl.BlockSpec((bB, N_NODES, W_OUT), lambda i: (i, 0, 0)),
        out_shape=jax.ShapeDtypeStruct((B, N_NODES, W_OUT), jnp.float32),
    )(xs, w1t, w2t, s_a, s_m, mmat)


def _bn_rows(y, g, be):
    mu = jnp.mean(y, axis=0, keepdims=True)
    va = jnp.mean((y - mu) * (y - mu), axis=0, keepdims=True)
    return (y - mu) / jnp.sqrt(va + 1e-5) * g + be


def _s3_body(d_ref, al_ref, ag_ref,
             lw1_ref, lb1_ref, lg1_ref, lbe1_ref,
             lw2_ref, lb2_ref, lg2_ref, lbe2_ref,
             gw1_ref, gb1_ref, gg1_ref, gbe1_ref,
             gw2_ref, gb2_ref, gg2_ref, gbe2_ref,
             c1e_ref, c1o_ref, cb1_ref, cg_ref, cbe_ref,
             c2_ref, cb2_ref, out_ref):
    delta = d_ref[...]

    def hops(A5, h3):
        A5b = jnp.broadcast_to(A5[None], (B, N_NODES, N_NODES))
        return lax.dot_general(
            A5b, h3, dimension_numbers=(((2,), (1,)), ((0,), (0,))),
            preferred_element_type=jnp.float32)

    def chain(a_ref, w1t, b1, g1, be1, w2t, b2, g2, be2):
        A = a_ref[...]
        A2 = jnp.dot(A, A, preferred_element_type=jnp.float32)
        A4 = jnp.dot(A2, A2, preferred_element_type=jnp.float32)
        A5 = jnp.dot(A4, A, preferred_element_type=jnp.float32)
        p = hops(A5, delta)
        y = jnp.dot(p.reshape(B * N_NODES, W_OUT), w1t,
                    preferred_element_type=jnp.float32) + b1
        y = _bn_rows(y, g1, be1)
        p2 = hops(A5, y.reshape(B, N_NODES, 16))
        z = jnp.dot(p2.reshape(B * N_NODES, 16), w2t,
                    preferred_element_type=jnp.float32) + b2
        z = _bn_rows(z, g2, be2)
        return jnp.mean(z.reshape(B, N_NODES, 4), axis=2)

    lf = chain(al_ref, lw1_ref[...], lb1_ref[...], lg1_ref[...], lbe1_ref[...],
               lw2_ref[...], lb2_ref[...], lg2_ref[...], lbe2_ref[...])
    gf = chain(ag_ref, gw1_ref[...], gb1_ref[...], gg1_ref[...], gbe1_ref[...],
               gw2_ref[...], gb2_ref[...], gg2_ref[...], gbe2_ref[...])

    h = (jnp.dot(lf, c1e_ref[...], preferred_element_type=jnp.float32)
         + jnp.dot(gf, c1o_ref[...], preferred_element_type=jnp.float32)
         + cb1_ref[...])
    h = _bn_rows(h, cg_ref[...], cbe_ref[...])
    logits = jnp.dot(h, c2_ref[...], preferred_element_type=jnp.float32) \
        + cb2_ref[...]
    mxl = jnp.max(logits, axis=1, keepdims=True)
    s = logits - mxl
    lse = jnp.log(jnp.sum(jnp.exp(s), axis=1, keepdims=True))
    out_ref[...] = s - lse


def _stage3(delta, a_l, a_g, wl, wg, wc):
    full = lambda s: pl.BlockSpec(s, lambda: tuple(0 for _ in s))
    args = [delta, a_l, a_g, *wl, *wg, *wc]
    specs = [full(a.shape) for a in args]
    return pl.pallas_call(
        _s3_body,
        in_specs=specs,
        out_specs=full((B, 2)),
        out_shape=jax.ShapeDtypeStruct((B, 2), jnp.float32),
    )(*args)


def kernel(x, edges, local_graph_weight, global_mask, params):
    xs = x.reshape(B, N_NODES, W_IN)

    loop = jnp.arange(N_NODES, dtype=jnp.int32)
    src_f = jnp.concatenate([edges[0].astype(jnp.int32), loop])
    dst_f = jnp.concatenate([edges[1].astype(jnp.int32), loop])
    wloc = jnp.concatenate([local_graph_weight,
                            jnp.ones((N_NODES,), jnp.float32)])
    awx = jnp.concatenate([params['adj_w'],
                           jnp.ones((8,), jnp.float32)])
    gidx = jnp.concatenate([jnp.asarray(_GIDX),
                            jnp.full((N_NODES,), 2080, jnp.int32)])
    mvec = jnp.concatenate([global_mask.reshape(-1),
                            jnp.ones((N_NODES,), jnp.float32)])
    a_l, a_g = _build_adj(src_f, dst_f, wloc, awx, gidx, mvec)
    a_l = a_l.reshape(N_NODES, N_NODES)
    a_g = a_g.reshape(N_NODES, N_NODES)

    w1t = params['ca_w1'].T
    w2t = params['ca_w2'].T
    ka = params['sa_w'][0, 0, 3, ::-1]
    km = params['sa_w'][0, 1, 3, ::-1]
    eyes = jnp.asarray(_EYES)
    s_a = jnp.einsum('k,kab->ab', ka, eyes)
    s_m = jnp.einsum('k,kab->ab', km, eyes)
    delta = _stage1(xs, w1t, w2t, s_a, s_m, jnp.asarray(_M_TOTAL))

    def row(v):
        return v.reshape(1, -1)
    wl = (params['ld_w'].T, row(params['ld_b']), row(params['ld_g']),
          row(params['ld_be']),
          params['ld1_w'].T, row(params['ld1_b']), row(params['ld1_g']),
          row(params['ld1_be']))
    wg = (params['d_w'].T, row(params['d_b']), row(params['d_g']),
          row(params['d_be']),
          params['d1_w'].T, row(params['d1_b']), row(params['d1_g']),
          row(params['d1_be']))
    wc = (params['cls_w1'][:, 0::2].T, params['cls_w1'][:, 1::2].T,
          row(params['cls_b1']), row(params['cls_g']), row(params['cls_be']),
          params['cls_w2'].T, row(params['cls_b2']))
    return _stage3(delta, a_l, a_g, wl, wg, wc)
